# baseline (speedup 1.0000x reference)
import functools

import jax
import jax.numpy as jnp
from jax import lax
from jax.experimental import pallas as pl
from jax.experimental.pallas import tpu as pltpu
from jax.experimental.pallas import tpu_sc as plsc

K = 8192
D = 256
B = 8
HW = 1024
N = B * HW
KT = 512
NKT = K // KT
COMMITMENT_COST = 0.25


def _vq_argmin_body(z_ref, e_ref, code_ref, loss_ref,
                    zz_ref, ee_ref, s2_ref, min_ref, arg_ref, acc_ref):
    b = pl.program_id(0)
    k = pl.program_id(1)

    z_blk = z_ref[0]

    @pl.when(k == 0)
    def _():
        zz_ref[...] = jnp.sum(z_blk * z_blk, axis=0, keepdims=True)

    @pl.when(k < NKT)
    def _():
        e_blk = e_ref[...]

        @pl.when(b == 0)
        def _():
            ee_ref[k] = jnp.sum(e_blk * e_blk, axis=1, keepdims=True)

        s2_ref[k % 2] = jnp.dot(e_blk + e_blk, z_blk,
                                preferred_element_type=jnp.float32)

    @pl.when(k > 0)
    def _():
        j = k - 1
        d = (zz_ref[...] - s2_ref[j % 2]) + ee_ref[j]

        dmin = jnp.min(d, axis=0, keepdims=True)
        amin = (jnp.argmin(d, axis=0).astype(jnp.float32)[None, :]
                + (j * KT).astype(jnp.float32))

        @pl.when(j == 0)
        def _():
            min_ref[...] = dmin
            arg_ref[...] = amin

        @pl.when(j > 0)
        def _():
            better = dmin < min_ref[...]
            arg_ref[...] = jnp.where(better, amin, arg_ref[...])
            min_ref[...] = jnp.minimum(dmin, min_ref[...])

        @pl.when(j == NKT - 1)
        def _():
            pos = arg_ref[0].astype(jnp.int32)
            rt = pos & (KT - 1)
            s = rt & 7
            r = rt >> 3
            rank = jnp.where(s == 1, 7, jnp.where(s == 2, 3, jnp.where(
                s == 3, 5, jnp.where(s == 4, 1, jnp.where(
                    s == 5, 6, jnp.where(
                        s == 6, 2, jnp.where(s == 7, 4, 0)))))))
            code_ref[...] = (pos - rt) + rank * (KT // 8) + r

            @pl.when(b == 0)
            def _():
                acc_ref[0, 0] = 0.0

            acc_ref[0, 0] += jnp.sum(min_ref[...])

            @pl.when(b == B - 1)
            def _():
                m = acc_ref[0, 0] / (N * D)
                loss_ref[0, 0] = m + m * COMMITMENT_COST


def _vq_argmin(z3, embedding):
    return pl.pallas_call(
        _vq_argmin_body,
        grid=(B, NKT + 1),
        in_specs=[
            pl.BlockSpec((1, D, HW), lambda b, k: (b, 0, 0)),
            pl.BlockSpec((KT, D), lambda b, k: (jnp.minimum(k, NKT - 1), 0)),
        ],
        out_specs=[
            pl.BlockSpec((HW,), lambda b, k: (b,)),
            pl.BlockSpec(memory_space=pltpu.SMEM),
        ],
        out_shape=[
            jax.ShapeDtypeStruct((N,), jnp.int32),
            jax.ShapeDtypeStruct((1, 1), jnp.float32),
        ],
        scratch_shapes=[
            pltpu.VMEM((1, HW), jnp.float32),
            pltpu.VMEM((NKT, KT, 1), jnp.float32),
            pltpu.VMEM((2, KT, HW), jnp.float32),
            pltpu.VMEM((1, HW), jnp.float32),
            pltpu.VMEM((1, HW), jnp.float32),
            pltpu.SMEM((1, 1), jnp.float32),
        ],
    )(z3, embedding)


def _make_sc_gather():
    info = plsc.get_sparse_core_info()
    nw = info.num_cores * info.num_subcores
    b_per_w = N // nw
    mesh = plsc.VectorSubcoreMesh(core_axis_name="c", subcore_axis_name="s")

    @functools.partial(
        pl.kernel, mesh=mesh,
        out_type=jax.ShapeDtypeStruct((N, D), jnp.float32),
        scratch_types=[
            pltpu.VMEM((b_per_w,), jnp.int32),
            pltpu.VMEM((b_per_w, D), jnp.float32),
            pltpu.SemaphoreType.DMA,
        ],
    )
    def gather(table_hbm, idx_hbm, out_hbm, idx_v, rows_v, sem):
        wid = lax.axis_index("s") * info.num_cores + lax.axis_index("c")
        base = wid * b_per_w
        pltpu.sync_copy(idx_hbm.at[pl.ds(base, b_per_w)], idx_v)
        pltpu.async_copy(table_hbm.at[idx_v], rows_v, sem).wait()
        pltpu.sync_copy(rows_v, out_hbm.at[pl.ds(base, b_per_w)])

    return gather


_SUBPRI = (0, 7, 3, 5, 1, 6, 2, 4)


def kernel(z, embedding):
    z3 = z.reshape(B, D, HW)
    e4 = embedding.reshape(NKT, 8, KT // 8, D)
    e_perm = e4[:, list(_SUBPRI), :, :].transpose(0, 2, 1, 3).reshape(K, D)
    code_flat, loss = _vq_argmin(z3, e_perm)
    zq_rows = _make_sc_gather()(embedding, code_flat)
    zq = zq_rows.reshape(B, 32, 32, D).transpose(0, 3, 1, 2)
    code = code_flat.reshape(B, 32, 32)
    return (zq, loss[0, 0], code)


# speedup vs baseline: 1.4011x; 1.4011x over previous
import functools

import jax
import jax.numpy as jnp
from jax import lax
from jax.experimental import pallas as pl
from jax.experimental.pallas import tpu as pltpu
from jax.experimental.pallas import tpu_sc as plsc

K = 8192
D = 256
B = 8
HW = 1024
N = B * HW
KT = 512
NKT = K // KT
COMMITMENT_COST = 0.25


def _vq_argmin_body(z_ref, e_ref, code_ref, loss_ref,
                    ee_ref, sa_ref, sb_ref, min_ref, arg_ref, acc_ref):
    b = pl.program_id(0)

    z_blk = z_ref[0]
    zz = jnp.sum(z_blk * z_blk, axis=0, keepdims=True)

    @pl.when(b == 0)
    def _():
        for t in range(NKT):
            e_blk = e_ref[pl.ds(t * KT, KT), :]
            ee_ref[t] = jnp.sum(e_blk * e_blk, axis=1, keepdims=True)

    def mm(t, out_ref):
        e_blk = e_ref[pl.ds(t * KT, KT), :]
        out_ref[...] = jnp.dot(e_blk + e_blk, z_blk,
                               preferred_element_type=jnp.float32)

    def post(t, s2_ref):
        d = (zz - s2_ref[...]) + ee_ref[t]
        dmin = jnp.min(d, axis=0, keepdims=True)
        amin = (jnp.argmin(d, axis=0).astype(jnp.float32)[None, :]
                + jnp.asarray(t * KT).astype(jnp.float32))
        better = dmin < min_ref[...]
        arg_ref[...] = jnp.where(better, amin, arg_ref[...])
        min_ref[...] = jnp.minimum(dmin, min_ref[...])

    min_ref[...] = jnp.full((1, HW), jnp.inf, jnp.float32)
    arg_ref[...] = jnp.zeros((1, HW), jnp.float32)
    mm(0, sa_ref)
    mm(1, sb_ref)

    def loop(i, _):
        ta = 2 * i
        post(ta, sa_ref)
        mm(ta + 2, sa_ref)
        post(ta + 1, sb_ref)
        mm(ta + 3, sb_ref)
        return 0

    lax.fori_loop(0, NKT // 2 - 1, loop, 0)
    post(NKT - 2, sa_ref)
    post(NKT - 1, sb_ref)

    pos = arg_ref[0].astype(jnp.int32)
    rt = pos & (KT - 1)
    s = rt & 7
    r = rt >> 3
    rank = jnp.where(s == 1, 7, jnp.where(s == 2, 3, jnp.where(
        s == 3, 5, jnp.where(s == 4, 1, jnp.where(
            s == 5, 6, jnp.where(s == 6, 2, jnp.where(s == 7, 4, 0)))))))
    code_ref[...] = (pos - rt) + rank * (KT // 8) + r

    @pl.when(b == 0)
    def _():
        acc_ref[0, 0] = 0.0

    acc_ref[0, 0] += jnp.sum(min_ref[...])

    @pl.when(b == B - 1)
    def _():
        m = acc_ref[0, 0] / (N * D)
        loss_ref[0, 0] = m + m * COMMITMENT_COST


def _vq_argmin(z3, embedding):
    return pl.pallas_call(
        _vq_argmin_body,
        grid=(B,),
        in_specs=[
            pl.BlockSpec((1, D, HW), lambda b: (b, 0, 0)),
            pl.BlockSpec((K, D), lambda b: (0, 0)),
        ],
        out_specs=[
            pl.BlockSpec((HW,), lambda b: (b,)),
            pl.BlockSpec(memory_space=pltpu.SMEM),
        ],
        out_shape=[
            jax.ShapeDtypeStruct((N,), jnp.int32),
            jax.ShapeDtypeStruct((1, 1), jnp.float32),
        ],
        scratch_shapes=[
            pltpu.VMEM((NKT, KT, 1), jnp.float32),
            pltpu.VMEM((KT, HW), jnp.float32),
            pltpu.VMEM((KT, HW), jnp.float32),
            pltpu.VMEM((1, HW), jnp.float32),
            pltpu.VMEM((1, HW), jnp.float32),
            pltpu.SMEM((1, 1), jnp.float32),
        ],
    )(z3, embedding)


def _make_sc_gather():
    info = plsc.get_sparse_core_info()
    nw = info.num_cores * info.num_subcores
    b_per_w = N // nw
    mesh = plsc.VectorSubcoreMesh(core_axis_name="c", subcore_axis_name="s")

    @functools.partial(
        pl.kernel, mesh=mesh,
        out_type=jax.ShapeDtypeStruct((N, D), jnp.float32),
        scratch_types=[
            pltpu.VMEM((b_per_w,), jnp.int32),
            pltpu.VMEM((b_per_w, D), jnp.float32),
            pltpu.SemaphoreType.DMA,
        ],
    )
    def gather(table_hbm, idx_hbm, out_hbm, idx_v, rows_v, sem):
        wid = lax.axis_index("s") * info.num_cores + lax.axis_index("c")
        base = wid * b_per_w
        pltpu.sync_copy(idx_hbm.at[pl.ds(base, b_per_w)], idx_v)
        pltpu.async_copy(table_hbm.at[idx_v], rows_v, sem).wait()
        pltpu.sync_copy(rows_v, out_hbm.at[pl.ds(base, b_per_w)])

    return gather


_SUBPRI = (0, 7, 3, 5, 1, 6, 2, 4)


def kernel(z, embedding):
    z3 = z.reshape(B, D, HW)
    e4 = embedding.reshape(NKT, 8, KT // 8, D)
    e_perm = e4[:, list(_SUBPRI), :, :].transpose(0, 2, 1, 3).reshape(K, D)
    code_flat, loss = _vq_argmin(z3, e_perm)
    zq_rows = _make_sc_gather()(embedding, code_flat)
    zq = zq_rows.reshape(B, 32, 32, D).transpose(0, 3, 1, 2)
    code = code_flat.reshape(B, 32, 32)
    return (zq, loss[0, 0], code)


# speedup vs baseline: 1.4866x; 1.0610x over previous
import functools

import jax
import jax.numpy as jnp
from jax import lax
from jax.experimental import pallas as pl
from jax.experimental.pallas import tpu as pltpu
from jax.experimental.pallas import tpu_sc as plsc

K = 8192
D = 256
B = 8
HW = 1024
N = B * HW
KT = 1024
NKT = K // KT
COMMITMENT_COST = 0.25


def _vq_argmin_body(z_ref, e_ref, code_ref, loss_ref,
                    ee_ref, sa_ref, sb_ref, min_ref, arg_ref, acc_ref):
    b = pl.program_id(0)

    z_blk = z_ref[0]
    zz = jnp.sum(z_blk * z_blk, axis=0, keepdims=True)

    @pl.when(b == 0)
    def _():
        for t in range(NKT):
            e_blk = e_ref[pl.ds(t * KT, KT), :]
            ee_ref[t] = jnp.sum(e_blk * e_blk, axis=1, keepdims=True)

    def mm(t, out_ref):
        e_blk = e_ref[pl.ds(t * KT, KT), :]
        out_ref[...] = jnp.dot(e_blk + e_blk, z_blk,
                               preferred_element_type=jnp.float32)

    def post(t, s2_ref):
        d = (zz - s2_ref[...]) + ee_ref[t]
        dmin = jnp.min(d, axis=0, keepdims=True)
        amin = (jnp.argmin(d, axis=0).astype(jnp.float32)[None, :]
                + jnp.asarray(t * KT).astype(jnp.float32))
        better = dmin < min_ref[...]
        arg_ref[...] = jnp.where(better, amin, arg_ref[...])
        min_ref[...] = jnp.minimum(dmin, min_ref[...])

    min_ref[...] = jnp.full((1, HW), jnp.inf, jnp.float32)
    arg_ref[...] = jnp.zeros((1, HW), jnp.float32)
    mm(0, sa_ref)
    mm(1, sb_ref)

    def loop(i, _):
        ta = 2 * i
        post(ta, sa_ref)
        mm(ta + 2, sa_ref)
        post(ta + 1, sb_ref)
        mm(ta + 3, sb_ref)
        return 0

    lax.fori_loop(0, NKT // 2 - 1, loop, 0)
    post(NKT - 2, sa_ref)
    post(NKT - 1, sb_ref)

    pos = arg_ref[0].astype(jnp.int32)
    rt = pos & (KT - 1)
    s = rt & 7
    r = rt >> 3
    rank = jnp.where(s == 1, 7, jnp.where(s == 2, 3, jnp.where(
        s == 3, 5, jnp.where(s == 4, 1, jnp.where(
            s == 5, 6, jnp.where(s == 6, 2, jnp.where(s == 7, 4, 0)))))))
    code_ref[...] = (pos - rt) + rank * (KT // 8) + r

    @pl.when(b == 0)
    def _():
        acc_ref[0, 0] = 0.0

    acc_ref[0, 0] += jnp.sum(min_ref[...])

    @pl.when(b == B - 1)
    def _():
        m = acc_ref[0, 0] / (N * D)
        loss_ref[0, 0] = m + m * COMMITMENT_COST


def _vq_argmin(z3, embedding):
    return pl.pallas_call(
        _vq_argmin_body,
        grid=(B,),
        in_specs=[
            pl.BlockSpec((1, D, HW), lambda b: (b, 0, 0)),
            pl.BlockSpec((K, D), lambda b: (0, 0)),
        ],
        out_specs=[
            pl.BlockSpec((HW,), lambda b: (b,)),
            pl.BlockSpec(memory_space=pltpu.SMEM),
        ],
        out_shape=[
            jax.ShapeDtypeStruct((N,), jnp.int32),
            jax.ShapeDtypeStruct((1, 1), jnp.float32),
        ],
        scratch_shapes=[
            pltpu.VMEM((NKT, KT, 1), jnp.float32),
            pltpu.VMEM((KT, HW), jnp.float32),
            pltpu.VMEM((KT, HW), jnp.float32),
            pltpu.VMEM((1, HW), jnp.float32),
            pltpu.VMEM((1, HW), jnp.float32),
            pltpu.SMEM((1, 1), jnp.float32),
        ],
    )(z3, embedding)


def _make_sc_gather():
    info = plsc.get_sparse_core_info()
    nw = info.num_cores * info.num_subcores
    b_per_w = N // nw
    mesh = plsc.VectorSubcoreMesh(core_axis_name="c", subcore_axis_name="s")

    @functools.partial(
        pl.kernel, mesh=mesh,
        out_type=jax.ShapeDtypeStruct((N, D), jnp.float32),
        scratch_types=[
            pltpu.VMEM((b_per_w,), jnp.int32),
            pltpu.VMEM((b_per_w, D), jnp.float32),
            pltpu.SemaphoreType.DMA,
        ],
    )
    def gather(table_hbm, idx_hbm, out_hbm, idx_v, rows_v, sem):
        wid = lax.axis_index("s") * info.num_cores + lax.axis_index("c")
        base = wid * b_per_w
        pltpu.sync_copy(idx_hbm.at[pl.ds(base, b_per_w)], idx_v)
        pltpu.async_copy(table_hbm.at[idx_v], rows_v, sem).wait()
        pltpu.sync_copy(rows_v, out_hbm.at[pl.ds(base, b_per_w)])

    return gather


_SUBPRI = (0, 7, 3, 5, 1, 6, 2, 4)


def kernel(z, embedding):
    z3 = z.reshape(B, D, HW)
    e4 = embedding.reshape(NKT, 8, KT // 8, D)
    e_perm = e4[:, list(_SUBPRI), :, :].transpose(0, 2, 1, 3).reshape(K, D)
    code_flat, loss = _vq_argmin(z3, e_perm)
    zq_rows = _make_sc_gather()(embedding, code_flat)
    zq = zq_rows.reshape(B, 32, 32, D).transpose(0, 3, 1, 2)
    code = code_flat.reshape(B, 32, 32)
    return (zq, loss[0, 0], code)


# speedup vs baseline: 1.8225x; 1.2259x over previous
import functools

import jax
import jax.numpy as jnp
from jax import lax
from jax.experimental import pallas as pl
from jax.experimental.pallas import tpu as pltpu
from jax.experimental.pallas import tpu_sc as plsc

K = 8192
D = 256
B = 8
HW = 1024
N = B * HW
KT = 2048
NKT = K // KT
COMMITMENT_COST = 0.25


def _vq_argmin_body(z_ref, e_ref, code_ref, loss_ref,
                    ee_ref, sa_ref, sb_ref, min_ref, arg_ref, acc_ref):
    b = pl.program_id(0)

    z_blk = z_ref[0]
    zz = jnp.sum(z_blk * z_blk, axis=0, keepdims=True)

    @pl.when(b == 0)
    def _():
        for t in range(NKT):
            e_blk = e_ref[pl.ds(t * KT, KT), :]
            ee_ref[t] = jnp.sum(e_blk * e_blk, axis=1, keepdims=True)

    def mm(t, out_ref):
        e_blk = e_ref[pl.ds(t * KT, KT), :]
        out_ref[...] = jnp.dot(e_blk + e_blk, z_blk,
                               preferred_element_type=jnp.float32)

    def post(t, s2_ref):
        d = (zz - s2_ref[...]) + ee_ref[t]
        dmin = jnp.min(d, axis=0, keepdims=True)
        amin = (jnp.argmin(d, axis=0).astype(jnp.float32)[None, :]
                + jnp.asarray(t * KT).astype(jnp.float32))
        better = dmin < min_ref[...]
        arg_ref[...] = jnp.where(better, amin, arg_ref[...])
        min_ref[...] = jnp.minimum(dmin, min_ref[...])

    min_ref[...] = jnp.full((1, HW), jnp.inf, jnp.float32)
    arg_ref[...] = jnp.zeros((1, HW), jnp.float32)
    mm(0, sa_ref)
    mm(1, sb_ref)

    def loop(i, _):
        ta = 2 * i
        post(ta, sa_ref)
        mm(ta + 2, sa_ref)
        post(ta + 1, sb_ref)
        mm(ta + 3, sb_ref)
        return 0

    lax.fori_loop(0, NKT // 2 - 1, loop, 0)
    post(NKT - 2, sa_ref)
    post(NKT - 1, sb_ref)

    pos = arg_ref[0].astype(jnp.int32)
    rt = pos & (KT - 1)
    s = rt & 7
    r = rt >> 3
    rank = jnp.where(s == 1, 7, jnp.where(s == 2, 3, jnp.where(
        s == 3, 5, jnp.where(s == 4, 1, jnp.where(
            s == 5, 6, jnp.where(s == 6, 2, jnp.where(s == 7, 4, 0)))))))
    code_ref[...] = (pos - rt) + rank * (KT // 8) + r

    @pl.when(b == 0)
    def _():
        acc_ref[0, 0] = 0.0

    acc_ref[0, 0] += jnp.sum(min_ref[...])

    @pl.when(b == B - 1)
    def _():
        m = acc_ref[0, 0] / (N * D)
        loss_ref[0, 0] = m + m * COMMITMENT_COST


def _vq_argmin(z3, embedding):
    return pl.pallas_call(
        _vq_argmin_body,
        grid=(B,),
        in_specs=[
            pl.BlockSpec((1, D, HW), lambda b: (b, 0, 0)),
            pl.BlockSpec((K, D), lambda b: (0, 0)),
        ],
        out_specs=[
            pl.BlockSpec((HW,), lambda b: (b,)),
            pl.BlockSpec(memory_space=pltpu.SMEM),
        ],
        out_shape=[
            jax.ShapeDtypeStruct((N,), jnp.int32),
            jax.ShapeDtypeStruct((1, 1), jnp.float32),
        ],
        scratch_shapes=[
            pltpu.VMEM((NKT, KT, 1), jnp.float32),
            pltpu.VMEM((KT, HW), jnp.float32),
            pltpu.VMEM((KT, HW), jnp.float32),
            pltpu.VMEM((1, HW), jnp.float32),
            pltpu.VMEM((1, HW), jnp.float32),
            pltpu.SMEM((1, 1), jnp.float32),
        ],
    )(z3, embedding)


def _make_sc_gather():
    info = plsc.get_sparse_core_info()
    nw = info.num_cores * info.num_subcores
    b_per_w = N // nw
    mesh = plsc.VectorSubcoreMesh(core_axis_name="c", subcore_axis_name="s")

    @functools.partial(
        pl.kernel, mesh=mesh,
        out_type=jax.ShapeDtypeStruct((N, D), jnp.float32),
        scratch_types=[
            pltpu.VMEM((b_per_w,), jnp.int32),
            pltpu.VMEM((b_per_w, D), jnp.float32),
            pltpu.SemaphoreType.DMA,
        ],
    )
    def gather(table_hbm, idx_hbm, out_hbm, idx_v, rows_v, sem):
        wid = lax.axis_index("s") * info.num_cores + lax.axis_index("c")
        base = wid * b_per_w
        pltpu.sync_copy(idx_hbm.at[pl.ds(base, b_per_w)], idx_v)
        pltpu.async_copy(table_hbm.at[idx_v], rows_v, sem).wait()
        pltpu.sync_copy(rows_v, out_hbm.at[pl.ds(base, b_per_w)])

    return gather


_SUBPRI = (0, 7, 3, 5, 1, 6, 2, 4)


def kernel(z, embedding):
    z3 = z.reshape(B, D, HW)
    e4 = embedding.reshape(NKT, 8, KT // 8, D)
    e_perm = e4[:, list(_SUBPRI), :, :].transpose(0, 2, 1, 3).reshape(K, D)
    code_flat, loss = _vq_argmin(z3, e_perm)
    zq_rows = _make_sc_gather()(embedding, code_flat)
    zq = zq_rows.reshape(B, 32, 32, D).transpose(0, 3, 1, 2)
    code = code_flat.reshape(B, 32, 32)
    return (zq, loss[0, 0], code)


# speedup vs baseline: 1.8497x; 1.0150x over previous
import functools

import jax
import jax.numpy as jnp
from jax import lax
from jax.experimental import pallas as pl
from jax.experimental.pallas import tpu as pltpu
from jax.experimental.pallas import tpu_sc as plsc

K = 8192
D = 256
B = 8
HW = 1024
N = B * HW
KT = 4096
NKT = K // KT
COMMITMENT_COST = 0.25


def _vq_argmin_body(z_ref, e_ref, code_ref, loss_ref,
                    ee_ref, sa_ref, sb_ref, min_ref, arg_ref, acc_ref):
    b = pl.program_id(0)

    z_blk = z_ref[0]
    zz = jnp.sum(z_blk * z_blk, axis=0, keepdims=True)

    @pl.when(b == 0)
    def _():
        for t in range(NKT):
            e_blk = e_ref[pl.ds(t * KT, KT), :]
            ee_ref[t] = jnp.sum(e_blk * e_blk, axis=1, keepdims=True)

    def mm(t, out_ref):
        e_blk = e_ref[pl.ds(t * KT, KT), :]
        out_ref[...] = jnp.dot(e_blk + e_blk, z_blk,
                               preferred_element_type=jnp.float32)

    def post(t, s2_ref):
        d = (zz - s2_ref[...]) + ee_ref[t]
        dmin = jnp.min(d, axis=0, keepdims=True)
        amin = (jnp.argmin(d, axis=0).astype(jnp.float32)[None, :]
                + jnp.asarray(t * KT).astype(jnp.float32))
        better = dmin < min_ref[...]
        arg_ref[...] = jnp.where(better, amin, arg_ref[...])
        min_ref[...] = jnp.minimum(dmin, min_ref[...])

    min_ref[...] = jnp.full((1, HW), jnp.inf, jnp.float32)
    arg_ref[...] = jnp.zeros((1, HW), jnp.float32)
    mm(0, sa_ref)
    mm(1, sb_ref)

    def loop(i, _):
        ta = 2 * i
        post(ta, sa_ref)
        mm(ta + 2, sa_ref)
        post(ta + 1, sb_ref)
        mm(ta + 3, sb_ref)
        return 0

    lax.fori_loop(0, NKT // 2 - 1, loop, 0)
    post(NKT - 2, sa_ref)
    post(NKT - 1, sb_ref)

    pos = arg_ref[0].astype(jnp.int32)
    rt = pos & (KT - 1)
    s = rt & 7
    r = rt >> 3
    rank = jnp.where(s == 1, 7, jnp.where(s == 2, 3, jnp.where(
        s == 3, 5, jnp.where(s == 4, 1, jnp.where(
            s == 5, 6, jnp.where(s == 6, 2, jnp.where(s == 7, 4, 0)))))))
    code_ref[...] = (pos - rt) + rank * (KT // 8) + r

    @pl.when(b == 0)
    def _():
        acc_ref[0, 0] = 0.0

    acc_ref[0, 0] += jnp.sum(min_ref[...])

    @pl.when(b == B - 1)
    def _():
        m = acc_ref[0, 0] / (N * D)
        loss_ref[0, 0] = m + m * COMMITMENT_COST


def _vq_argmin(z3, embedding):
    return pl.pallas_call(
        _vq_argmin_body,
        grid=(B,),
        in_specs=[
            pl.BlockSpec((1, D, HW), lambda b: (b, 0, 0)),
            pl.BlockSpec((K, D), lambda b: (0, 0)),
        ],
        out_specs=[
            pl.BlockSpec((HW,), lambda b: (b,)),
            pl.BlockSpec(memory_space=pltpu.SMEM),
        ],
        out_shape=[
            jax.ShapeDtypeStruct((N,), jnp.int32),
            jax.ShapeDtypeStruct((1, 1), jnp.float32),
        ],
        scratch_shapes=[
            pltpu.VMEM((NKT, KT, 1), jnp.float32),
            pltpu.VMEM((KT, HW), jnp.float32),
            pltpu.VMEM((KT, HW), jnp.float32),
            pltpu.VMEM((1, HW), jnp.float32),
            pltpu.VMEM((1, HW), jnp.float32),
            pltpu.SMEM((1, 1), jnp.float32),
        ],
    )(z3, embedding)


def _make_sc_gather():
    info = plsc.get_sparse_core_info()
    nw = info.num_cores * info.num_subcores
    b_per_w = N // nw
    mesh = plsc.VectorSubcoreMesh(core_axis_name="c", subcore_axis_name="s")

    @functools.partial(
        pl.kernel, mesh=mesh,
        out_type=jax.ShapeDtypeStruct((N, D), jnp.float32),
        scratch_types=[
            pltpu.VMEM((b_per_w,), jnp.int32),
            pltpu.VMEM((b_per_w, D), jnp.float32),
            pltpu.SemaphoreType.DMA,
        ],
    )
    def gather(table_hbm, idx_hbm, out_hbm, idx_v, rows_v, sem):
        wid = lax.axis_index("s") * info.num_cores + lax.axis_index("c")
        base = wid * b_per_w
        pltpu.sync_copy(idx_hbm.at[pl.ds(base, b_per_w)], idx_v)
        pltpu.async_copy(table_hbm.at[idx_v], rows_v, sem).wait()
        pltpu.sync_copy(rows_v, out_hbm.at[pl.ds(base, b_per_w)])

    return gather


_SUBPRI = (0, 7, 3, 5, 1, 6, 2, 4)


def kernel(z, embedding):
    z3 = z.reshape(B, D, HW)
    e4 = embedding.reshape(NKT, 8, KT // 8, D)
    e_perm = e4[:, list(_SUBPRI), :, :].transpose(0, 2, 1, 3).reshape(K, D)
    code_flat, loss = _vq_argmin(z3, e_perm)
    zq_rows = _make_sc_gather()(embedding, code_flat)
    zq = zq_rows.reshape(B, 32, 32, D).transpose(0, 3, 1, 2)
    code = code_flat.reshape(B, 32, 32)
    return (zq, loss[0, 0], code)
